# Initial kernel scaffold; baseline (speedup 1.0000x reference)
import jax
import jax.numpy as jnp
from jax.experimental import pallas as pl


def kernel(x, train_mask, edge_index_u, edge_weight_u, wenc, conv_w, wdec):
    raise NotImplementedError("write your pallas kernel here")


# speedup vs baseline: 2.4255x; 2.4255x over previous
import dataclasses
import functools

import jax
import jax.numpy as jnp
from jax import lax
from jax.experimental import pallas as pl
from jax.experimental.pallas import tpu as pltpu
from jax.experimental.pallas import tpu_sc as plsc

N = 10000
N_PAD = 10240
N_HALF = 5120
ITEMS = 2000
EMB = 500
EMBP = 512
CH = 128
NCH = 4
E = 160000
E_T = E // 16
E_W = E // 32
BLK = 80
NBLK = E_T // BLK
ROWS_T = N_HALF // 16
DROWS_T = N_PAD // 16
AGG_R = N_HALF + 16
REG_C = 0.001 / 3.0

_HIGH = lax.Precision.HIGHEST


def _frsqrt(p):
    i = plsc.bitcast(p, jnp.int32)
    i = jnp.int32(0x5F3759DF) - (i >> 1)
    y = plsc.bitcast(i, jnp.float32)
    for _ in range(3):
        y = y * (1.5 - 0.5 * p * y * y)
    return y


def _enc_body(x_ref, m_ref, w_ref, xt_ref, z4_ref):
    xt = x_ref[...] * m_ref[...]
    xt_ref[...] = xt
    z = jnp.maximum(
        jnp.dot(xt, w_ref[...], preferred_element_type=jnp.float32,
                precision=_HIGH), 0.0)
    for c in range(NCH):
        z4_ref[c] = z[:, c * CH:(c + 1) * CH]


_enc = pl.pallas_call(
    _enc_body,
    grid=(25,),
    in_specs=[
        pl.BlockSpec((400, ITEMS), lambda i: (i, 0)),
        pl.BlockSpec((400, ITEMS), lambda i: (i, 0)),
        pl.BlockSpec((ITEMS, EMBP), lambda i: (0, 0)),
    ],
    out_specs=[
        pl.BlockSpec((400, ITEMS), lambda i: (i, 0)),
        pl.BlockSpec((NCH, 400, CH), lambda i: (0, i, 0)),
    ],
    out_shape=[
        jax.ShapeDtypeStruct((N, ITEMS), jnp.float32),
        jax.ShapeDtypeStruct((NCH, N, CH), jnp.float32),
    ],
)


def _dec_body(cv_ref, wd_ref, we_ref, cw4_ref, rl_ref):
    cw = jnp.dot(cv_ref[...], wd_ref[...], preferred_element_type=jnp.float32,
                 precision=_HIGH)
    for c in range(NCH):
        cw4_ref[c] = cw[c * CH:(c + 1) * CH, :]
    we = we_ref[...]
    cv = cv_ref[...]
    wd = wd_ref[...]
    rl_ref[0, 0] = REG_C * (jnp.sum(we * we) + jnp.sum(cv * cv)
                            + jnp.sum(wd * wd))


_dec_w = pl.pallas_call(
    _dec_body,
    out_specs=[
        pl.BlockSpec((NCH, CH, ITEMS), lambda: (0, 0, 0)),
        pl.BlockSpec(memory_space=pltpu.SMEM),
    ],
    out_shape=[
        jax.ShapeDtypeStruct((NCH, CH, ITEMS), jnp.float32),
        jax.ShapeDtypeStruct((1, 1), jnp.float32),
    ],
)


def _pred_body(a4_ref, cw4_ref, o_ref):
    acc = jnp.dot(a4_ref[0], cw4_ref[0], preferred_element_type=jnp.float32,
                  precision=_HIGH)
    for c in range(1, NCH):
        acc = acc + jnp.dot(a4_ref[c], cw4_ref[c],
                            preferred_element_type=jnp.float32,
                            precision=_HIGH)
    o_ref[...] = acc


_pred = pl.pallas_call(
    _pred_body,
    grid=(25,),
    in_specs=[
        pl.BlockSpec((NCH, 400, CH), lambda i: (0, i, 0)),
        pl.BlockSpec((NCH, CH, ITEMS), lambda i: (0, 0, 0)),
    ],
    out_specs=pl.BlockSpec((400, ITEMS), lambda i: (i, 0)),
    out_shape=jax.ShapeDtypeStruct((N, ITEMS), jnp.float32),
)


_mesh = plsc.VectorSubcoreMesh(core_axis_name="c", subcore_axis_name="s")

_sc_params = pltpu.CompilerParams()
if "needs_layout_passes" in pltpu.CompilerParams.__dataclass_fields__:
    _sc_params = dataclasses.replace(_sc_params, needs_layout_passes=False)


@functools.partial(
    pl.kernel,
    out_type=jax.ShapeDtypeStruct((E,), jnp.float32),
    mesh=_mesh,
    compiler_params=_sc_params,
    scratch_types=[
        pltpu.VMEM((NBLK, BLK), jnp.int32),
        pltpu.VMEM((NBLK, BLK), jnp.float32),
        pltpu.VMEM((N_PAD,), jnp.float32),
        pltpu.VMEM((E_W,), jnp.int32),
        pltpu.VMEM((E_W,), jnp.int32),
        pltpu.VMEM((E_W,), jnp.float32),
        pltpu.VMEM((E_W,), jnp.float32),
        pltpu.VMEM((DROWS_T,), jnp.float32),
        pltpu.VMEM_SHARED((N_PAD,), jnp.float32),
    ],
)
def _sc_norm(dst3_hbm, ew3_hbm, srcf_hbm, dstf_hbm, ewf_hbm, norm_hbm,
             dstv, eww, degv, srcn, dstn, ewn, nout, zb, degsh):
    cid = lax.axis_index("c")
    sid = lax.axis_index("s")
    pltpu.sync_copy(dst3_hbm.at[sid], dstv)
    pltpu.sync_copy(ew3_hbm.at[sid], eww)

    @pl.loop(0, DROWS_T, step=16)
    def _(i):
        zb[pl.ds(i, 16)] = jnp.zeros((16,), jnp.float32)

    pltpu.sync_copy(zb, degsh.at[pl.ds(sid * DROWS_T, DROWS_T)])
    plsc.subcore_barrier()

    @pl.loop(0, NBLK)
    def _(j):
        pltpu.sync_copy(eww.at[j], degsh.at[dstv.at[j]], add=True)

    plsc.subcore_barrier()
    pltpu.sync_copy(degsh, degv)

    wid = cid * 16 + sid
    base = wid * E_W
    pltpu.sync_copy(srcf_hbm.at[pl.ds(base, E_W)], srcn)
    pltpu.sync_copy(dstf_hbm.at[pl.ds(base, E_W)], dstn)
    pltpu.sync_copy(ewf_hbm.at[pl.ds(base, E_W)], ewn)

    def _norm16(e):
        s16 = srcn[pl.ds(e, 16)]
        d16 = dstn[pl.ds(e, 16)]
        degs = plsc.load_gather(degv, [s16])
        degd = plsc.load_gather(degv, [d16])
        r = _frsqrt(degs * degd + 1e-12)
        nout[pl.ds(e, 16)] = ewn[pl.ds(e, 16)] * r

    @pl.loop(0, E_W - 8, step=16)
    def _(e):
        _norm16(e)

    _norm16(E_W - 16)
    pltpu.sync_copy(nout, norm_hbm.at[pl.ds(base, E_W)])


@functools.partial(
    pl.kernel,
    out_type=jax.ShapeDtypeStruct((NCH * N_PAD, CH), jnp.float32),
    mesh=_mesh,
    compiler_params=_sc_params,
    scratch_types=[
        pltpu.VMEM((NBLK, BLK), jnp.int32),
        pltpu.VMEM((NBLK, BLK), jnp.int32),
        pltpu.VMEM((NBLK, BLK), jnp.int32),
        pltpu.VMEM((E_T,), jnp.float32),
        pltpu.VMEM((BLK, CH), jnp.float32),
        pltpu.VMEM((CH, CH), jnp.float32),
        pltpu.VMEM_SHARED((AGG_R, CH), jnp.float32),
    ],
)
def _sc_agg(src3_hbm, dst3_hbm, normf_hbm, zflat_hbm, aggf_hbm,
            srcv, srcv2, dstv, nrmv, gbuf, zb, aggsh):
    cid = lax.axis_index("c")
    sid = lax.axis_index("s")
    pltpu.sync_copy(src3_hbm.at[sid], srcv)
    pltpu.sync_copy(dst3_hbm.at[sid], dstv)
    pltpu.sync_copy(normf_hbm.at[pl.ds(sid * E_T, E_T)], nrmv)

    @pl.loop(0, CH)
    def _(r):
        for k in range(0, CH, 16):
            zb[r, pl.ds(k, 16)] = jnp.zeros((16,), jnp.float32)

    nbase = cid * N_HALF
    trash = N_HALF + sid

    @pl.loop(0, NBLK)
    def _(j):
        for k in range(0, BLK, 16):
            d = dstv[j, pl.ds(k, 16)]
            m = (d >= nbase) & (d < nbase + N_HALF)
            dstv[j, pl.ds(k, 16)] = jnp.where(m, d - nbase, trash)

    for ch in range(NCH):
        zoff = ch * N

        for b in range(ROWS_T // CH):
            pltpu.sync_copy(zb, aggsh.at[pl.ds(sid * ROWS_T + b * CH, CH)])
        pltpu.sync_copy(zb.at[pl.ds(0, ROWS_T % CH)],
                        aggsh.at[pl.ds(sid * ROWS_T + (ROWS_T // CH) * CH,
                                       ROWS_T % CH)])
        pltpu.sync_copy(zb.at[pl.ds(0, 1)], aggsh.at[pl.ds(trash, 1)])

        @pl.loop(0, NBLK)
        def _(j):
            for k in range(0, BLK, 16):
                srcv2[j, pl.ds(k, 16)] = srcv[j, pl.ds(k, 16)] + zoff

        plsc.subcore_barrier()

        @pl.loop(0, NBLK)
        def _(j):
            pltpu.sync_copy(zflat_hbm.at[srcv2.at[j]], gbuf)

            @pl.loop(0, BLK, step=16)
            def _(r0):
                n16 = nrmv[pl.ds(j * BLK + r0, 16)]
                for r in range(16):
                    nj = n16[r]
                    for k in range(0, CH, 16):
                        gbuf[r0 + r, pl.ds(k, 16)] = (
                            gbuf[r0 + r, pl.ds(k, 16)] * nj)

            pltpu.sync_copy(gbuf, aggsh.at[dstv.at[j]], add=True)

        plsc.subcore_barrier()
        pltpu.sync_copy(
            aggsh.at[pl.ds(sid * ROWS_T, ROWS_T)],
            aggf_hbm.at[pl.ds(ch * N_PAD + cid * N_HALF + sid * ROWS_T,
                              ROWS_T)])


def kernel(x, train_mask, edge_index_u, edge_weight_u, wenc, conv_w, wdec):
    src = edge_index_u[0].astype(jnp.int32)
    dst = edge_index_u[1].astype(jnp.int32)
    ew = edge_weight_u.astype(jnp.float32)
    src3 = src.reshape(16, NBLK, BLK)
    dst3 = dst.reshape(16, NBLK, BLK)
    ew3 = ew.reshape(16, NBLK, BLK)

    wenc_p = jnp.pad(wenc, ((0, 0), (0, EMBP - EMB)))
    conv_p = jnp.pad(conv_w, ((0, EMBP - EMB), (0, EMBP - EMB)))
    wdec_p = jnp.pad(wdec, ((0, EMBP - EMB), (0, 0)))

    xt, z4 = _enc(x, train_mask, wenc_p)
    cw4, rl = _dec_w(conv_p, wdec_p, wenc_p)
    norm = _sc_norm(dst3, ew3, src, dst, ew)
    aggf = _sc_agg(src3, dst3, norm, z4.reshape(NCH * N, CH))
    pred = _pred(aggf.reshape(NCH, N_PAD, CH), cw4)
    return (xt, pred, rl.reshape(()))
